# baseline (speedup 1.0000x reference)
import functools

import jax
import jax.numpy as jnp
from jax import lax
from jax.experimental import pallas as pl
from jax.experimental.pallas import tpu as pltpu
from jax.experimental.pallas import tpu_sc as plsc

EPS = 1e-08
N = 3276800
NUM_CORES = 2
NUM_SUBCORES = 16
NW = NUM_CORES * NUM_SUBCORES
PER_W = N // NW
CHUNK = 10240
NCHUNK = PER_W // CHUNK
LANES = 16
VECS = CHUNK // LANES

_mesh = plsc.VectorSubcoreMesh(core_axis_name="c", subcore_axis_name="s")


@functools.partial(
    pl.kernel,
    mesh=_mesh,
    out_type=jax.ShapeDtypeStruct((N,), jnp.float32),
    compiler_params=pltpu.CompilerParams(needs_layout_passes=False),
    scratch_types=[
        pltpu.VMEM((LANES,), jnp.float32),
        pltpu.VMEM((LANES,), jnp.float32),
        pltpu.VMEM((LANES,), jnp.float32),
        pltpu.VMEM((LANES,), jnp.float32),
        pltpu.VMEM((CHUNK,), jnp.float32),
        pltpu.VMEM((CHUNK,), jnp.int32),
        pltpu.VMEM((CHUNK,), jnp.float32),
    ],
)
def _range_norm_sc(x_hbm, g_hbm, mins_hbm, maxs_hbm, out_hbm,
                   mins_v, maxs_v, a_v, b_v, x_v, g_v, o_v):
    wid = lax.axis_index("s") * NUM_CORES + lax.axis_index("c")
    base = wid * PER_W

    pltpu.sync_copy(mins_hbm, mins_v)
    pltpu.sync_copy(maxs_hbm, maxs_v)
    m = mins_v[...]
    a = (1.0 - 2.0 * EPS) / (maxs_v[...] - m)
    a_v[...] = a
    b_v[...] = EPS - m * a

    def chunk_body(ci, carry):
        off = base + ci * CHUNK
        pltpu.sync_copy(x_hbm.at[pl.ds(off, CHUNK)], x_v)
        pltpu.sync_copy(g_hbm.at[pl.ds(off, CHUNK)], g_v)

        def vec_body(vi, c2):
            s = pl.ds(vi * LANES, LANES)
            idx = g_v[s] - 1
            av = plsc.load_gather(a_v, [idx])
            bv = plsc.load_gather(b_v, [idx])
            o_v[s] = x_v[s] * av + bv
            return c2

        lax.fori_loop(0, VECS, vec_body, 0)
        pltpu.sync_copy(o_v, out_hbm.at[pl.ds(off, CHUNK)])
        return carry

    lax.fori_loop(0, NCHUNK, chunk_body, 0)


def kernel(x, group, mins, maxs):
    return _range_norm_sc(x, group, mins, maxs)
